# baseline (device time: 15008 ns/iter reference)
import jax
import jax.numpy as jnp
from jax import lax
from jax.experimental import pallas as pl
from jax.experimental.pallas import tpu as pltpu

N_DEV = 16
N_STEPS = 8


def kernel(x, w_mat):
    m_per, k = x.shape
    n = w_mat.shape[1]
    n_per = n // N_DEV
    m_out = m_per * N_DEV

    def body(x_ref, w_ref, out_ref, xb_ref, y_blk_ref, send_sems, recv_sems):
        g = pl.program_id(0)
        my = lax.axis_index("i")
        barrier = pltpu.get_barrier_semaphore()

        @pl.when(g == 0)
        def _():
            for d in range(1, N_DEV):
                j = lax.rem(my + d, N_DEV)
                pl.semaphore_signal(
                    barrier, inc=1,
                    device_id=(j,), device_id_type=pl.DeviceIdType.MESH,
                )
            xb_ref[:, :] = x_ref[:, :].astype(jnp.bfloat16)

        wb = w_ref[:, :].astype(jnp.bfloat16)
        yb = lax.dot(xb_ref[:, :], wb, preferred_element_type=jnp.float32)
        yb = (yb * jax.nn.sigmoid(yb)).astype(jnp.bfloat16)
        y_blk_ref[2 * g, :, :] = yb[:, :n_per]
        y_blk_ref[2 * g + 1, :, :] = yb[:, n_per:]

        @pl.when(g == N_STEPS - 1)
        def _():
            pl.semaphore_wait(barrier, N_DEV - 1)

            out_ref[pl.ds(my * m_per, m_per), :] = y_blk_ref[my, :, :]

            for d in range(1, N_DEV):
                j = lax.rem(my + d, N_DEV)
                rdma = pltpu.make_async_remote_copy(
                    src_ref=y_blk_ref.at[j],
                    dst_ref=out_ref.at[pl.ds(my * m_per, m_per), :],
                    send_sem=send_sems.at[d],
                    recv_sem=recv_sems.at[my],
                    device_id=(j,),
                    device_id_type=pl.DeviceIdType.MESH,
                )
                rdma.start()

            for d in range(1, N_DEV):
                i = lax.rem(my - d + N_DEV, N_DEV)
                recv = pltpu.make_async_remote_copy(
                    src_ref=y_blk_ref.at[0],
                    dst_ref=out_ref.at[pl.ds(i * m_per, m_per), :],
                    send_sem=send_sems.at[0],
                    recv_sem=recv_sems.at[i],
                    device_id=(i,),
                    device_id_type=pl.DeviceIdType.MESH,
                )
                recv.wait_recv()

            for d in range(1, N_DEV):
                snd = pltpu.make_async_remote_copy(
                    src_ref=y_blk_ref.at[0],
                    dst_ref=out_ref.at[pl.ds(0, m_per), :],
                    send_sem=send_sems.at[d],
                    recv_sem=recv_sems.at[0],
                    device_id=(0,),
                    device_id_type=pl.DeviceIdType.MESH,
                )
                snd.wait_send()

    return pl.pallas_call(
        body,
        grid=(N_STEPS,),
        out_shape=jax.ShapeDtypeStruct((m_out, n_per), jnp.bfloat16),
        in_specs=[
            pl.BlockSpec((m_per, k), lambda g: (0, 0)),
            pl.BlockSpec((k, 2 * n_per), lambda g: (0, g)),
        ],
        out_specs=pl.BlockSpec((m_out, n_per), lambda g: (0, 0)),
        scratch_shapes=[
            pltpu.VMEM((m_per, k), jnp.bfloat16),
            pltpu.VMEM((N_DEV, m_per, n_per), jnp.bfloat16),
            pltpu.SemaphoreType.DMA((N_DEV,)),
            pltpu.SemaphoreType.DMA((N_DEV,)),
        ],
        compiler_params=pltpu.CompilerParams(collective_id=0),
    )(x, w_mat)
